# baseline (speedup 1.0000x reference)
import functools

import jax
import jax.numpy as jnp
from jax.experimental import pallas as pl
from jax.experimental.pallas import tpu as pltpu

B = 4
IN_DIM = 1024
OUT_DIM = 1024
ATTN = 96
H = 3
DH = 32
NBINS = 8
RANGE = 3.0
TOPK = 32
MIN_TAU = 0.05
L = 1024
BLK_I = 1024

CENTERS = [(-RANGE + i * (2.0 * RANGE / (NBINS - 1))) for i in range(NBINS)]
DELTA = 2.0 * RANGE / (NBINS - 1)
INV_DELTA = 1.0 / (DELTA + 1e-6)


def _bf16_matvec(a, w_row):
    return jnp.sum(a.astype(jnp.bfloat16).astype(jnp.float32)
                   * w_row.astype(jnp.bfloat16).astype(jnp.float32),
                   axis=1, keepdims=True)


def _ln_lanes(x, g, b, eps=1e-5):
    mu = jnp.mean(x, axis=-1, keepdims=True)
    var = jnp.mean((x - mu) ** 2, axis=-1, keepdims=True)
    return (x - mu) / jnp.sqrt(var + eps) * g + b


def _prew_body(x_ref, pre_W_ref, pre_b_ref, v_ref):
    v_ref[...] = jnp.dot(x_ref[...], pre_W_ref[...],
                         preferred_element_type=jnp.float32) + pre_b_ref[...]


def _prologue_body(v2_ref, emb_W_ref, emb_b_ref,
                   ln1_g_ref, ln1_b_ref, Wq_ref, Wk_ref, Wv_ref,
                   Wq1_ref, Wk1_ref, tau_W_ref, tau_c_ref,
                   tok_ref, vv_ref, uq_ref, uk_ref, tau_ref):
    tok = v2_ref[...] * emb_W_ref[...] + emb_b_ref[...]
    h1 = _ln_lanes(tok, ln1_g_ref[...], ln1_b_ref[...])
    tok_ref[...] = tok
    vv_ref[...] = jnp.dot(h1, Wv_ref[...], preferred_element_type=jnp.float32)
    q2 = jnp.dot(h1, Wq_ref[...], preferred_element_type=jnp.float32)
    k2 = jnp.dot(h1, Wk_ref[...], preferred_element_type=jnp.float32)
    uq_ref[...] = jnp.concatenate(
        [_bf16_matvec(q2[:, h * DH:(h + 1) * DH], Wq1_ref[...])
         for h in range(H)], axis=1)
    uk_ref[...] = jnp.concatenate(
        [_bf16_matvec(k2[:, h * DH:(h + 1) * DH], Wk1_ref[...])
         for h in range(H)], axis=1)
    u = _bf16_matvec(h1, tau_W_ref[...])
    s = jnp.zeros_like(u)
    for m in range(NBINS):
        hb = jnp.maximum(1.0 - jnp.abs(u - CENTERS[m]) / (DELTA + 1e-6), 0.0)
        s = s + hb * tau_c_ref[0, m]
    tau_ref[...] = jax.nn.softplus(s) + MIN_TAU


def _main_body(uq_ref, uk_ref, tau_ref, coeff_ref, vv_ref, ctx_ref):
    uq = jnp.reshape(uq_ref[...], (BLK_I, 1))
    uk = jnp.reshape(uk_ref[...], (1, L))
    diff = uq - uk
    K = jnp.zeros_like(diff)
    for m in range(NBINS):
        hb = jnp.maximum(1.0 - jnp.abs(diff - CENTERS[m]) / (DELTA + 1e-6), 0.0)
        K = K + hb * coeff_ref[0, 0, m]
    K = K / (jnp.reshape(tau_ref[...], (BLK_I, 1)) + 1e-6)
    cols = jax.lax.broadcasted_iota(jnp.int32, (BLK_I, L), 1)
    m0 = jnp.max(K, axis=1, keepdims=True)

    def step(_, carry):
        work, W, denom = carry
        m = jnp.max(work, axis=1, keepdims=True)
        cand = jnp.where(work == m, cols, L)
        sel = jnp.min(cand, axis=1, keepdims=True)
        onehot = cols == sel
        w = jnp.exp(m - m0)
        W = W + jnp.where(onehot, w, 0.0)
        denom = denom + w
        work = jnp.where(onehot, -jnp.inf, work)
        return work, W, denom

    _, W, denom = jax.lax.fori_loop(
        0, TOPK, step,
        (K, jnp.zeros_like(K), jnp.zeros_like(m0)))
    attnW = W * (1.0 / denom)
    vv = jnp.reshape(vv_ref[...], (L, DH))
    ctx_ref[...] = jnp.reshape(
        jnp.dot(attnW, vv, precision="highest",
                preferred_element_type=jnp.float32),
        (1, 1, BLK_I, DH))


def _epilogue_body(tok_ref, ctx_ref, Wo_ref, ln2_g_ref, ln2_b_ref,
                   W1_ref, b1_ref, W2_ref, b2_ref, po_W_ref, po_b_ref,
                   scalar_ref):
    tok = tok_ref[...] + jnp.dot(ctx_ref[...], Wo_ref[...],
                                 preferred_element_type=jnp.float32)
    h2 = _ln_lanes(tok, ln2_g_ref[...], ln2_b_ref[...])
    a = jnp.dot(h2, W1_ref[...],
                preferred_element_type=jnp.float32) + b1_ref[...]
    f = 0.5 * a * (1.0 + jax.lax.erf(a * (2.0 ** -0.5)))
    f = jnp.dot(f, W2_ref[...], preferred_element_type=jnp.float32) + b2_ref[...]
    tok = tok + f
    scalar_ref[...] = _bf16_matvec(tok, po_W_ref[...]) + po_b_ref[0, 0]


def _final_ln_body(s4_ref, x_ref, lnf_g_ref, lnf_b_ref, out_ref):
    out_ref[...] = _ln_lanes(s4_ref[...] + x_ref[...],
                             lnf_g_ref[...], lnf_b_ref[...])


@jax.jit
def kernel(x, pre_W, pre_b, emb_W, emb_b, ln1_g, ln1_b, Wq, Wk, Wv, Wq1, Wk1,
           kern_coeff, Wo, tau_W, tau_coeff, ln2_g, ln2_b, ffn_W1, ffn_b1,
           ffn_W2, ffn_b2, po_W, po_b, lnf_g, lnf_b):
    f32 = jnp.float32
    row = lambda t: t.reshape(1, -1)

    v = pl.pallas_call(
        _prew_body,
        out_shape=jax.ShapeDtypeStruct((B, OUT_DIM), f32),
    )(x, pre_W, row(pre_b))
    v2 = v.reshape(B * L, 1)

    tok2, vv2, uq3, uk3, tau2 = pl.pallas_call(
        _prologue_body,
        out_shape=[
            jax.ShapeDtypeStruct((B * L, ATTN), f32),
            jax.ShapeDtypeStruct((B * L, ATTN), f32),
            jax.ShapeDtypeStruct((B * L, H), f32),
            jax.ShapeDtypeStruct((B * L, H), f32),
            jax.ShapeDtypeStruct((B * L, 1), f32),
        ],
    )(v2, row(emb_W), row(emb_b), row(ln1_g), row(ln1_b),
      Wq, Wk, Wv, row(Wq1), row(Wk1), row(tau_W), tau_coeff)

    uqA = uq3.reshape(B, L, H).transpose(0, 2, 1).reshape(B * H, L, 1)
    ukA = uk3.reshape(B, L, H).transpose(0, 2, 1).reshape(B * H, 1, L)
    tauA = tau2.reshape(B, L, 1)
    coeffA = kern_coeff.reshape(H, 1, NBINS)
    vvA = vv2.reshape(B, L, H, DH).transpose(0, 2, 1, 3)

    nblk = L // BLK_I
    ctx4 = pl.pallas_call(
        _main_body,
        grid=(B * H, nblk),
        in_specs=[
            pl.BlockSpec((1, BLK_I, 1), lambda bh, ib: (bh, ib, 0)),
            pl.BlockSpec((1, 1, L), lambda bh, ib: (bh, 0, 0)),
            pl.BlockSpec((1, BLK_I, 1), lambda bh, ib: (bh // H, ib, 0)),
            pl.BlockSpec((1, 1, NBINS), lambda bh, ib: (bh % H, 0, 0)),
            pl.BlockSpec((1, 1, L, DH), lambda bh, ib: (bh // H, bh % H, 0, 0)),
        ],
        out_specs=pl.BlockSpec(
            (1, 1, BLK_I, DH), lambda bh, ib: (bh // H, bh % H, ib, 0)),
        out_shape=jax.ShapeDtypeStruct((B, H, L, DH), f32),
        compiler_params=pltpu.CompilerParams(
            dimension_semantics=("parallel", "parallel")),
    )(uqA, ukA, tauA, coeffA, vvA)
    ctx2 = ctx4.transpose(0, 2, 1, 3).reshape(B * L, ATTN)

    scalar = pl.pallas_call(
        _epilogue_body,
        out_shape=jax.ShapeDtypeStruct((B * L, 1), f32),
    )(tok2, ctx2, Wo, row(ln2_g), row(ln2_b), ffn_W1, row(ffn_b1), ffn_W2,
      row(ffn_b2), row(po_W), po_b.reshape(1, 1))

    out = pl.pallas_call(
        _final_ln_body,
        out_shape=jax.ShapeDtypeStruct((B, IN_DIM), f32),
    )(scalar.reshape(B, IN_DIM), x, row(lnf_g), row(lnf_b))
    return out


# speedup vs baseline: 1.0159x; 1.0159x over previous
import jax
import jax.numpy as jnp
from jax.experimental import pallas as pl
from jax.experimental.pallas import tpu as pltpu

B = 4
IN_DIM = 1024
OUT_DIM = 1024
ATTN = 96
H = 3
DH = 32
NBINS = 8
RANGE = 3.0
TOPK = 32
MIN_TAU = 0.05
L = 1024
BLK_I = 512

CENTERS = [(-RANGE + i * (2.0 * RANGE / (NBINS - 1))) for i in range(NBINS)]
DELTA = 2.0 * RANGE / (NBINS - 1)


def _bf16_matvec(a, w_row):
    return jnp.sum(a.astype(jnp.bfloat16).astype(jnp.float32)
                   * w_row.astype(jnp.bfloat16).astype(jnp.float32),
                   axis=1, keepdims=True)


def _ln_lanes(x, g, b, eps=1e-5):
    mu = jnp.mean(x, axis=-1, keepdims=True)
    var = jnp.mean((x - mu) ** 2, axis=-1, keepdims=True)
    return (x - mu) / jnp.sqrt(var + eps) * g + b


def _prew_body(x_ref, pre_W_ref, pre_b_ref, v_ref):
    v_ref[...] = jnp.dot(x_ref[...], pre_W_ref[...],
                         preferred_element_type=jnp.float32) + pre_b_ref[...]


def _prologue_body(v2_ref, emb_W_ref, emb_b_ref,
                   ln1_g_ref, ln1_b_ref, Wq_ref, Wk_ref, Wv_ref,
                   Wq1_ref, Wk1_ref, tau_W_ref, tau_c_ref,
                   tok_ref, vv_ref, uq_ref, uk_ref, tau_ref):
    tok = v2_ref[...] * emb_W_ref[...] + emb_b_ref[...]
    h1 = _ln_lanes(tok, ln1_g_ref[...], ln1_b_ref[...])
    tok_ref[...] = tok
    vv_ref[...] = jnp.dot(h1, Wv_ref[...], preferred_element_type=jnp.float32)
    q2 = jnp.dot(h1, Wq_ref[...], preferred_element_type=jnp.float32)
    k2 = jnp.dot(h1, Wk_ref[...], preferred_element_type=jnp.float32)
    uq_ref[...] = jnp.concatenate(
        [_bf16_matvec(q2[:, h * DH:(h + 1) * DH], Wq1_ref[...])
         for h in range(H)], axis=1)
    uk_ref[...] = jnp.concatenate(
        [_bf16_matvec(k2[:, h * DH:(h + 1) * DH], Wk1_ref[...])
         for h in range(H)], axis=1)
    u = _bf16_matvec(h1, tau_W_ref[...])
    s = jnp.zeros_like(u)
    for m in range(NBINS):
        hb = jnp.maximum(1.0 - jnp.abs(u - CENTERS[m]) / (DELTA + 1e-6), 0.0)
        s = s + hb * tau_c_ref[0, m]
    tau_ref[...] = jax.nn.softplus(s) + MIN_TAU


def _main_body(uq_ref, uk_ref, tau_ref, coeff_ref, vv_ref, ctx_ref):
    uq = jnp.reshape(uq_ref[...], (BLK_I, 1))
    uk = jnp.reshape(uk_ref[...], (1, L))
    diff = uq - uk
    K = jnp.zeros_like(diff)
    for m in range(NBINS):
        hb = jnp.maximum(1.0 - jnp.abs(diff - CENTERS[m]) / (DELTA + 1e-6), 0.0)
        K = K + hb * coeff_ref[0, 0, m]
    K = K / (jnp.reshape(tau_ref[...], (BLK_I, 1)) + 1e-6)
    cols = jax.lax.broadcasted_iota(jnp.int32, (BLK_I, L), 1)
    m0 = jnp.max(K, axis=1, keepdims=True)

    def step(_, carry):
        work, W, denom = carry
        m = jnp.max(work, axis=1, keepdims=True)
        cand = jnp.where(work == m, cols, L)
        sel = jnp.min(cand, axis=1, keepdims=True)
        onehot = cols == sel
        w = jnp.exp(m - m0)
        W = W + jnp.where(onehot, w, 0.0)
        denom = denom + w
        work = jnp.where(onehot, -jnp.inf, work)
        return work, W, denom

    _, W, denom = jax.lax.fori_loop(
        0, TOPK, step,
        (K, jnp.zeros_like(K), jnp.zeros_like(m0)))
    attnW = W * (1.0 / denom)
    vv = jnp.reshape(vv_ref[...], (L, DH))
    ctx_ref[...] = jnp.reshape(
        jnp.dot(attnW, vv, precision="highest",
                preferred_element_type=jnp.float32),
        (1, 1, BLK_I, DH))


def _epilogue_body(tok_ref, ctx_ref, Wo_ref, ln2_g_ref, ln2_b_ref,
                   W1_ref, b1_ref, W2_ref, b2_ref, po_W_ref, po_b_ref,
                   scalar_ref):
    tok = tok_ref[...] + jnp.dot(ctx_ref[...], Wo_ref[...],
                                 preferred_element_type=jnp.float32)
    h2 = _ln_lanes(tok, ln2_g_ref[...], ln2_b_ref[...])
    a = jnp.dot(h2, W1_ref[...],
                preferred_element_type=jnp.float32) + b1_ref[...]
    f = 0.5 * a * (1.0 + jax.lax.erf(a * (2.0 ** -0.5)))
    f = jnp.dot(f, W2_ref[...], preferred_element_type=jnp.float32) + b2_ref[...]
    tok = tok + f
    scalar_ref[...] = _bf16_matvec(tok, po_W_ref[...]) + po_b_ref[0, 0]


def _final_ln_body(s4_ref, x_ref, lnf_g_ref, lnf_b_ref, out_ref):
    out_ref[...] = _ln_lanes(s4_ref[...] + x_ref[...],
                             lnf_g_ref[...], lnf_b_ref[...])


@jax.jit
def kernel(x, pre_W, pre_b, emb_W, emb_b, ln1_g, ln1_b, Wq, Wk, Wv, Wq1, Wk1,
           kern_coeff, Wo, tau_W, tau_coeff, ln2_g, ln2_b, ffn_W1, ffn_b1,
           ffn_W2, ffn_b2, po_W, po_b, lnf_g, lnf_b):
    f32 = jnp.float32
    row = lambda t: t.reshape(1, -1)

    v = pl.pallas_call(
        _prew_body,
        out_shape=jax.ShapeDtypeStruct((B, OUT_DIM), f32),
    )(x, pre_W, row(pre_b))
    v2 = v.reshape(B * L, 1)

    tok2, vv2, uq3, uk3, tau2 = pl.pallas_call(
        _prologue_body,
        out_shape=[
            jax.ShapeDtypeStruct((B * L, ATTN), f32),
            jax.ShapeDtypeStruct((B * L, ATTN), f32),
            jax.ShapeDtypeStruct((B * L, H), f32),
            jax.ShapeDtypeStruct((B * L, H), f32),
            jax.ShapeDtypeStruct((B * L, 1), f32),
        ],
    )(v2, row(emb_W), row(emb_b), row(ln1_g), row(ln1_b),
      Wq, Wk, Wv, row(Wq1), row(Wk1), row(tau_W), tau_coeff)

    uqA = uq3.reshape(B, L, H).transpose(0, 2, 1).reshape(B * H, L, 1)
    ukA = uk3.reshape(B, L, H).transpose(0, 2, 1).reshape(B * H, 1, L)
    tauA = tau2.reshape(B, L, 1)
    coeffA = kern_coeff.reshape(H, 1, NBINS)
    vvA = vv2.reshape(B, L, H, DH).transpose(0, 2, 1, 3)

    nblk = L // BLK_I
    ctx4 = pl.pallas_call(
        _main_body,
        grid=(B * H, nblk),
        in_specs=[
            pl.BlockSpec((1, BLK_I, 1), lambda bh, ib: (bh, ib, 0)),
            pl.BlockSpec((1, 1, L), lambda bh, ib: (bh, 0, 0)),
            pl.BlockSpec((1, BLK_I, 1), lambda bh, ib: (bh // H, ib, 0)),
            pl.BlockSpec((1, 1, NBINS), lambda bh, ib: (bh % H, 0, 0)),
            pl.BlockSpec((1, 1, L, DH), lambda bh, ib: (bh // H, bh % H, 0, 0)),
        ],
        out_specs=pl.BlockSpec(
            (1, 1, BLK_I, DH), lambda bh, ib: (bh // H, bh % H, ib, 0)),
        out_shape=jax.ShapeDtypeStruct((B, H, L, DH), f32),
        compiler_params=pltpu.CompilerParams(
            dimension_semantics=("parallel", "parallel")),
    )(uqA, ukA, tauA, coeffA, vvA)
    ctx2 = ctx4.transpose(0, 2, 1, 3).reshape(B * L, ATTN)

    scalar = pl.pallas_call(
        _epilogue_body,
        out_shape=jax.ShapeDtypeStruct((B * L, 1), f32),
    )(tok2, ctx2, Wo, row(ln2_g), row(ln2_b), ffn_W1, row(ffn_b1), ffn_W2,
      row(ffn_b2), row(po_W), po_b.reshape(1, 1))

    out = pl.pallas_call(
        _final_ln_body,
        out_shape=jax.ShapeDtypeStruct((B, IN_DIM), f32),
    )(scalar.reshape(B, IN_DIM), x, row(lnf_g), row(lnf_b))
    return out
